# Initial kernel scaffold; baseline (speedup 1.0000x reference)
import jax
import jax.numpy as jnp
from jax.experimental import pallas as pl


def kernel(input_tensor, table):
    raise NotImplementedError("write your pallas kernel here")


# speedup vs baseline: 4.0848x; 4.0848x over previous
import functools

import jax
import jax.numpy as jnp
from jax import lax
from jax.experimental import pallas as pl
from jax.experimental.pallas import tpu as pltpu
from jax.experimental.pallas import tpu_sc as plsc

VOCAB = 100000
EMBED_DIM = 64
BATCH = 4096
HIST = 50

TOTAL = BATCH * HIST
NUM_CORES = 2
NUM_SUBCORES = 16
NW = NUM_CORES * NUM_SUBCORES
BPW = TOTAL // NW
CHUNK = 128
NCHUNK = BPW // CHUNK

_mesh = plsc.VectorSubcoreMesh(core_axis_name="c", subcore_axis_name="s")


@functools.partial(
    pl.kernel,
    out_type=jax.ShapeDtypeStruct((TOTAL, EMBED_DIM), jnp.float32),
    mesh=_mesh,
    scratch_types=[
        pltpu.VMEM((BPW,), jnp.int32),
        pltpu.VMEM((CHUNK, EMBED_DIM), jnp.float32),
        pltpu.SemaphoreType.DMA,
    ],
    compiler_params=pltpu.CompilerParams(use_tc_tiling_on_sc=False),
)
def _embed_gather(table_hbm, idx_hbm, out_hbm, idx_v, rows, gsem):
    wid = lax.axis_index("s") * NUM_CORES + lax.axis_index("c")
    base = wid * BPW
    pltpu.sync_copy(idx_hbm.at[pl.ds(base, BPW)], idx_v)

    def body(c, carry):
        off = pl.multiple_of(c * CHUNK, CHUNK)
        pltpu.async_copy(table_hbm.at[idx_v.at[pl.ds(off, CHUNK)]], rows, gsem).wait()
        pltpu.sync_copy(rows, out_hbm.at[pl.ds(base + off, CHUNK)])
        return carry

    lax.fori_loop(0, NCHUNK, body, 0)


def kernel(input_tensor, table):
    idx = input_tensor.reshape(-1).astype(jnp.int32)
    out = _embed_gather(table, idx)
    return out.reshape(BATCH, HIST, EMBED_DIM)


# speedup vs baseline: 4.5628x; 1.1170x over previous
import functools

import jax
import jax.numpy as jnp
from jax import lax
from jax.experimental import pallas as pl
from jax.experimental.pallas import tpu as pltpu
from jax.experimental.pallas import tpu_sc as plsc

VOCAB = 100000
EMBED_DIM = 64
BATCH = 4096
HIST = 50

TOTAL = BATCH * HIST
NUM_CORES = 2
NUM_SUBCORES = 16
NW = NUM_CORES * NUM_SUBCORES
BPW = TOTAL // NW
CHUNK = 128
NCHUNK = BPW // CHUNK

_mesh = plsc.VectorSubcoreMesh(core_axis_name="c", subcore_axis_name="s")


@functools.partial(
    pl.kernel,
    out_type=jax.ShapeDtypeStruct((TOTAL, EMBED_DIM), jnp.float32),
    mesh=_mesh,
    scratch_types=[
        pltpu.VMEM((BPW,), jnp.int32),
        pltpu.VMEM((CHUNK, EMBED_DIM), jnp.float32),
        pltpu.VMEM((CHUNK, EMBED_DIM), jnp.float32),
        pltpu.SemaphoreType.DMA,
        pltpu.SemaphoreType.DMA,
        pltpu.SemaphoreType.DMA,
        pltpu.SemaphoreType.DMA,
    ],
    compiler_params=pltpu.CompilerParams(use_tc_tiling_on_sc=False),
)
def _embed_gather(table_hbm, idx_hbm, out_hbm, idx_v, rows0, rows1,
                  g0, g1, s0, s1):
    wid = lax.axis_index("s") * NUM_CORES + lax.axis_index("c")
    base = wid * BPW
    pltpu.sync_copy(idx_hbm.at[pl.ds(base, BPW)], idx_v)

    rows = (rows0, rows1)
    gsem = (g0, g1)
    ssem = (s0, s1)

    def gather(n, b):
        off = pl.multiple_of(n * CHUNK, CHUNK)
        pltpu.async_copy(table_hbm.at[idx_v.at[pl.ds(off, CHUNK)]],
                         rows[b], gsem[b])

    def store(n, b):
        off = pl.multiple_of(n * CHUNK, CHUNK)
        pltpu.async_copy(rows[b], out_hbm.at[pl.ds(base + off, CHUNK)],
                         ssem[b])

    gather(0, 0)
    gather(1, 1)

    def body(i, carry):
        for b in range(2):
            n = 2 * i + b
            pltpu.make_async_copy(table_hbm.at[idx_v.at[pl.ds(0, CHUNK)]],
                                  rows[b], gsem[b]).wait()
            store(n, b)
            pltpu.make_async_copy(rows[b],
                                  out_hbm.at[pl.ds(0, CHUNK)],
                                  ssem[b]).wait()

            @pl.when(n + 2 < NCHUNK)
            def _():
                gather(n + 2, b)

        return carry

    lax.fori_loop(0, NCHUNK // 2, body, 0)


def kernel(input_tensor, table):
    idx = input_tensor.reshape(-1).astype(jnp.int32)
    out = _embed_gather(table, idx)
    return out.reshape(BATCH, HIST, EMBED_DIM)


# speedup vs baseline: 4.6715x; 1.0238x over previous
import functools

import jax
import jax.numpy as jnp
from jax import lax
from jax.experimental import pallas as pl
from jax.experimental.pallas import tpu as pltpu
from jax.experimental.pallas import tpu_sc as plsc

VOCAB = 100000
EMBED_DIM = 64
BATCH = 4096
HIST = 50

TOTAL = BATCH * HIST
NUM_CORES = 2
NUM_SUBCORES = 16
NW = NUM_CORES * NUM_SUBCORES
BPW = TOTAL // NW
CHUNK = 400
NCHUNK = BPW // CHUNK

_mesh = plsc.VectorSubcoreMesh(core_axis_name="c", subcore_axis_name="s")


@functools.partial(
    pl.kernel,
    out_type=jax.ShapeDtypeStruct((TOTAL, EMBED_DIM), jnp.float32),
    mesh=_mesh,
    scratch_types=[
        pltpu.VMEM((BPW,), jnp.int32),
        pltpu.VMEM((CHUNK, EMBED_DIM), jnp.float32),
        pltpu.VMEM((CHUNK, EMBED_DIM), jnp.float32),
        pltpu.SemaphoreType.DMA,
        pltpu.SemaphoreType.DMA,
        pltpu.SemaphoreType.DMA,
        pltpu.SemaphoreType.DMA,
    ],
    compiler_params=pltpu.CompilerParams(use_tc_tiling_on_sc=False),
)
def _embed_gather(table_hbm, idx_hbm, out_hbm, idx_v, rows0, rows1,
                  g0, g1, s0, s1):
    wid = lax.axis_index("s") * NUM_CORES + lax.axis_index("c")
    base = wid * BPW
    pltpu.sync_copy(idx_hbm.at[pl.ds(base, BPW)], idx_v)

    rows = (rows0, rows1)
    gsem = (g0, g1)
    ssem = (s0, s1)

    def gather(n, b):
        off = pl.multiple_of(n * CHUNK, CHUNK)
        pltpu.async_copy(table_hbm.at[idx_v.at[pl.ds(off, CHUNK)]],
                         rows[b], gsem[b])

    def store(n, b):
        off = pl.multiple_of(n * CHUNK, CHUNK)
        pltpu.async_copy(rows[b], out_hbm.at[pl.ds(base + off, CHUNK)],
                         ssem[b])

    gather(0, 0)
    gather(1, 1)

    def body(i, carry):
        for b in range(2):
            n = 2 * i + b
            pltpu.make_async_copy(table_hbm.at[idx_v.at[pl.ds(0, CHUNK)]],
                                  rows[b], gsem[b]).wait()
            store(n, b)
            pltpu.make_async_copy(rows[b],
                                  out_hbm.at[pl.ds(0, CHUNK)],
                                  ssem[b]).wait()

            @pl.when(n + 2 < NCHUNK)
            def _():
                gather(n + 2, b)

        return carry

    lax.fori_loop(0, NCHUNK // 2, body, 0)


def kernel(input_tensor, table):
    idx = input_tensor.reshape(-1).astype(jnp.int32)
    out = _embed_gather(table, idx)
    return out.reshape(BATCH, HIST, EMBED_DIM)


# speedup vs baseline: 4.9170x; 1.0526x over previous
import functools

import jax
import jax.numpy as jnp
from jax import lax
from jax.experimental import pallas as pl
from jax.experimental.pallas import tpu as pltpu
from jax.experimental.pallas import tpu_sc as plsc

VOCAB = 100000
EMBED_DIM = 64
BATCH = 4096
HIST = 50

TOTAL = BATCH * HIST
NUM_CORES = 2
NUM_SUBCORES = 16
NW = NUM_CORES * NUM_SUBCORES
BPW = TOTAL // NW
CHUNK = 400
NCHUNK = BPW // CHUNK

_mesh = plsc.VectorSubcoreMesh(core_axis_name="c", subcore_axis_name="s")


@functools.partial(
    pl.kernel,
    out_type=jax.ShapeDtypeStruct((TOTAL, EMBED_DIM), jnp.float32),
    mesh=_mesh,
    scratch_types=[
        pltpu.VMEM((BPW,), jnp.int32),
        pltpu.VMEM((CHUNK, EMBED_DIM), jnp.float32),
        pltpu.VMEM((CHUNK, EMBED_DIM), jnp.float32),
        pltpu.SemaphoreType.DMA,
        pltpu.SemaphoreType.DMA,
        pltpu.SemaphoreType.DMA,
        pltpu.SemaphoreType.DMA,
    ],
    compiler_params=pltpu.CompilerParams(use_tc_tiling_on_sc=False),
)
def _embed_gather(table_hbm, idx_hbm, out_hbm, idx_v, rows0, rows1,
                  g0, g1, s0, s1):
    wid = lax.axis_index("s") * NUM_CORES + lax.axis_index("c")
    base = wid * BPW
    pltpu.sync_copy(idx_hbm.at[pl.ds(base, BPW)], idx_v)

    rows = (rows0, rows1)
    gsem = (g0, g1)
    ssem = (s0, s1)

    def gather(n, b):
        off = pl.multiple_of(n * CHUNK, CHUNK)
        pltpu.async_copy(table_hbm.at[idx_v.at[pl.ds(off, CHUNK)]],
                         rows[b], gsem[b])

    def store(n, b):
        off = pl.multiple_of(n * CHUNK, CHUNK)
        pltpu.async_copy(rows[b], out_hbm.at[pl.ds(base + off, CHUNK)],
                         ssem[b])

    gather(0, 0)
    gather(1, 1)

    def body(i, carry):
        for b in range(2):
            n = 2 * i + b
            pltpu.make_async_copy(table_hbm.at[idx_v.at[pl.ds(0, CHUNK)]],
                                  rows[b], gsem[b]).wait()
            store(n, b)
            pltpu.make_async_copy(rows[b],
                                  out_hbm.at[pl.ds(0, CHUNK)],
                                  ssem[b]).wait()

            @pl.when(n + 2 < NCHUNK)
            def _():
                gather(n + 2, b)

        return carry

    lax.fori_loop(0, NCHUNK // 2, body, 0)


def kernel(input_tensor, table):
    idxt = input_tensor.T.reshape(-1)
    out = _embed_gather(table, idxt)
    return out.reshape(HIST, BATCH, EMBED_DIM).transpose(1, 0, 2)


# speedup vs baseline: 6.1982x; 1.2606x over previous
import functools

import jax
import jax.numpy as jnp
from jax import lax
from jax.experimental import pallas as pl
from jax.experimental.pallas import tpu as pltpu
from jax.experimental.pallas import tpu_sc as plsc

VOCAB = 100000
EMBED_DIM = 64
PADDED = 128
BATCH = 4096
HIST = 50

TOTAL = BATCH * HIST
NUM_CORES = 2
NUM_SUBCORES = 16
NW = NUM_CORES * NUM_SUBCORES
BPW = TOTAL // NW
CHUNK = 400
NCHUNK = BPW // CHUNK

_mesh = plsc.VectorSubcoreMesh(core_axis_name="c", subcore_axis_name="s")


@functools.partial(
    pl.kernel,
    out_type=jax.ShapeDtypeStruct((TOTAL, PADDED), jnp.float32),
    mesh=_mesh,
    scratch_types=[
        pltpu.VMEM((BPW,), jnp.int32),
        pltpu.VMEM((CHUNK, PADDED), jnp.float32),
        pltpu.VMEM((CHUNK, PADDED), jnp.float32),
        pltpu.SemaphoreType.DMA,
        pltpu.SemaphoreType.DMA,
        pltpu.SemaphoreType.DMA,
        pltpu.SemaphoreType.DMA,
    ],
    compiler_params=pltpu.CompilerParams(use_tc_tiling_on_sc=False),
)
def _embed_gather(table_hbm, idx_hbm, out_hbm, idx_v, rows0, rows1,
                  g0, g1, s0, s1):
    wid = lax.axis_index("s") * NUM_CORES + lax.axis_index("c")
    base = wid * BPW
    pltpu.sync_copy(idx_hbm.at[pl.ds(base, BPW)], idx_v)

    rows = (rows0, rows1)
    gsem = (g0, g1)
    ssem = (s0, s1)

    def gather(n, b):
        off = pl.multiple_of(n * CHUNK, CHUNK)
        pltpu.async_copy(table_hbm.at[idx_v.at[pl.ds(off, CHUNK)]],
                         rows[b], gsem[b])

    def store(n, b):
        off = pl.multiple_of(n * CHUNK, CHUNK)
        pltpu.async_copy(rows[b], out_hbm.at[pl.ds(base + off, CHUNK)],
                         ssem[b])

    gather(0, 0)
    gather(1, 1)

    def body(i, carry):
        for b in range(2):
            n = 2 * i + b
            pltpu.make_async_copy(table_hbm.at[idx_v.at[pl.ds(0, CHUNK)]],
                                  rows[b], gsem[b]).wait()
            store(n, b)
            pltpu.make_async_copy(rows[b],
                                  out_hbm.at[pl.ds(0, CHUNK)],
                                  ssem[b]).wait()

            @pl.when(n + 2 < NCHUNK)
            def _():
                gather(n + 2, b)

        return carry

    lax.fori_loop(0, NCHUNK // 2, body, 0)


def kernel(input_tensor, table):
    tpad = jnp.pad(table, ((0, 0), (0, PADDED - EMBED_DIM)))
    idxt = input_tensor.T.reshape(-1)
    out = _embed_gather(tpad, idxt)
    out = out.reshape(HIST, BATCH, PADDED)[:, :, :EMBED_DIM]
    return out.transpose(1, 0, 2)


# speedup vs baseline: 6.7733x; 1.0928x over previous
import functools

import jax
import jax.numpy as jnp
from jax import lax
from jax.experimental import pallas as pl
from jax.experimental.pallas import tpu as pltpu
from jax.experimental.pallas import tpu_sc as plsc

VOCAB = 100000
EMBED_DIM = 64
PADDED = 128
BATCH = 4096
HIST = 50

TOTAL = BATCH * HIST
NUM_CORES = 2
NUM_SUBCORES = 16
NW = NUM_CORES * NUM_SUBCORES
BPW = TOTAL // NW
CHUNK = 400
NCHUNK = BPW // CHUNK

_mesh = plsc.VectorSubcoreMesh(core_axis_name="c", subcore_axis_name="s")


@functools.partial(
    pl.kernel,
    out_type=jax.ShapeDtypeStruct((TOTAL, PADDED), jnp.float32),
    mesh=_mesh,
    scratch_types=[
        pltpu.VMEM((BPW,), jnp.int32),
        pltpu.VMEM((CHUNK, PADDED), jnp.float32),
        pltpu.VMEM((CHUNK, PADDED), jnp.float32),
        pltpu.SemaphoreType.DMA,
        pltpu.SemaphoreType.DMA,
        pltpu.SemaphoreType.DMA,
        pltpu.SemaphoreType.DMA,
    ],
    compiler_params=pltpu.CompilerParams(use_tc_tiling_on_sc=False),
)
def _embed_gather(table_hbm, idx_hbm, out_hbm, idx_v, rows0, rows1,
                  g0, g1, s0, s1):
    wid = lax.axis_index("s") * NUM_CORES + lax.axis_index("c")
    base = wid * BPW
    pltpu.sync_copy(idx_hbm.at[pl.ds(base, BPW)], idx_v)

    rows = (rows0, rows1)
    gsem = (g0, g1)
    ssem = (s0, s1)

    def gather(n, b):
        off = pl.multiple_of(n * CHUNK, CHUNK)
        pltpu.async_copy(table_hbm.at[idx_v.at[pl.ds(off, CHUNK)]],
                         rows[b], gsem[b])

    def store(n, b):
        off = pl.multiple_of(n * CHUNK, CHUNK)
        pltpu.async_copy(
            rows[b].at[:, pl.ds(0, EMBED_DIM)],
            out_hbm.at[pl.ds(base + off, CHUNK), pl.ds(0, EMBED_DIM)],
            ssem[b])

    gather(0, 0)
    gather(1, 1)

    def body(i, carry):
        for b in range(2):
            n = 2 * i + b
            pltpu.make_async_copy(table_hbm.at[idx_v.at[pl.ds(0, CHUNK)]],
                                  rows[b], gsem[b]).wait()
            store(n, b)
            pltpu.make_async_copy(
                rows[b].at[:, pl.ds(0, EMBED_DIM)],
                out_hbm.at[pl.ds(0, CHUNK), pl.ds(0, EMBED_DIM)],
                ssem[b]).wait()

            @pl.when(n + 2 < NCHUNK)
            def _():
                gather(n + 2, b)

        return carry

    lax.fori_loop(0, NCHUNK // 2, body, 0)


def kernel(input_tensor, table):
    tpad = jnp.pad(table, ((0, 0), (0, PADDED - EMBED_DIM)))
    idxt = input_tensor.T.reshape(-1)
    out = _embed_gather(tpad, idxt)
    out = out.reshape(HIST, BATCH, PADDED)[:, :, :EMBED_DIM]
    return out.transpose(1, 0, 2)


# speedup vs baseline: 7.3678x; 1.0878x over previous
import functools

import jax
import jax.numpy as jnp
from jax import lax
from jax.experimental import pallas as pl
from jax.experimental.pallas import tpu as pltpu
from jax.experimental.pallas import tpu_sc as plsc

VOCAB = 100000
EMBED_DIM = 64
PADDED = 128
BATCH = 4096
HIST = 50

TOTAL = BATCH * HIST
NUM_CORES = 2
NUM_SUBCORES = 16
NW = NUM_CORES * NUM_SUBCORES
BPW = TOTAL // NW
CHUNK = 400
NCHUNK = BPW // CHUNK

_mesh = plsc.VectorSubcoreMesh(core_axis_name="c", subcore_axis_name="s")


@functools.partial(
    pl.kernel,
    out_type=jax.ShapeDtypeStruct((TOTAL, PADDED), jnp.float32),
    mesh=_mesh,
    scratch_types=[
        pltpu.VMEM((BPW,), jnp.int32),
        pltpu.VMEM((CHUNK, EMBED_DIM), jnp.float32),
        pltpu.VMEM((CHUNK, EMBED_DIM), jnp.float32),
        pltpu.SemaphoreType.DMA,
        pltpu.SemaphoreType.DMA,
        pltpu.SemaphoreType.DMA,
        pltpu.SemaphoreType.DMA,
    ],
    compiler_params=pltpu.CompilerParams(use_tc_tiling_on_sc=False),
)
def _embed_gather(table_hbm, idx_hbm, out_hbm, idx_v, rows0, rows1,
                  g0, g1, s0, s1):
    wid = lax.axis_index("s") * NUM_CORES + lax.axis_index("c")
    base = wid * BPW
    pltpu.sync_copy(idx_hbm.at[pl.ds(base, BPW)], idx_v)

    rows = (rows0, rows1)
    gsem = (g0, g1)
    ssem = (s0, s1)

    def gather(n, b):
        off = pl.multiple_of(n * CHUNK, CHUNK)
        pltpu.async_copy(table_hbm.at[idx_v.at[pl.ds(off, CHUNK)]],
                         rows[b], gsem[b])

    def store(n, b):
        off = pl.multiple_of(n * CHUNK, CHUNK)
        pltpu.async_copy(
            rows[b],
            out_hbm.at[pl.ds(base + off, CHUNK), pl.ds(0, EMBED_DIM)],
            ssem[b])

    gather(0, 0)
    gather(1, 1)

    def body(i, carry):
        for b in range(2):
            n = 2 * i + b
            pltpu.make_async_copy(table_hbm.at[idx_v.at[pl.ds(0, CHUNK)]],
                                  rows[b], gsem[b]).wait()
            store(n, b)
            pltpu.make_async_copy(
                rows[b],
                out_hbm.at[pl.ds(0, CHUNK), pl.ds(0, EMBED_DIM)],
                ssem[b]).wait()

            @pl.when(n + 2 < NCHUNK)
            def _():
                gather(n + 2, b)

        return carry

    lax.fori_loop(0, NCHUNK // 2, body, 0)


def kernel(input_tensor, table):
    idxt = input_tensor.T.reshape(-1)
    out = _embed_gather(table, idxt)
    out = out.reshape(HIST, BATCH, PADDED)[:, :, :EMBED_DIM]
    return out.transpose(1, 0, 2)


# speedup vs baseline: 7.7303x; 1.0492x over previous
import functools

import jax
import jax.numpy as jnp
from jax import lax
from jax.experimental import pallas as pl
from jax.experimental.pallas import tpu as pltpu
from jax.experimental.pallas import tpu_sc as plsc

VOCAB = 100000
EMBED_DIM = 64
PADDED = 128
BATCH = 4096
HIST = 50

TOTAL = BATCH * HIST
NUM_CORES = 2
NUM_SUBCORES = 16
NW = NUM_CORES * NUM_SUBCORES
BPW = TOTAL // NW
CHUNK = 400
NCHUNK = BPW // CHUNK

_mesh = plsc.VectorSubcoreMesh(core_axis_name="c", subcore_axis_name="s")


@functools.partial(
    pl.kernel,
    out_type=jax.ShapeDtypeStruct((TOTAL, PADDED), jnp.float32),
    mesh=_mesh,
    scratch_types=[
        pltpu.VMEM((BPW,), jnp.int32),
        pltpu.VMEM((CHUNK, EMBED_DIM), jnp.float32),
        pltpu.VMEM((CHUNK, EMBED_DIM), jnp.float32),
        pltpu.SemaphoreType.DMA,
        pltpu.SemaphoreType.DMA,
        pltpu.SemaphoreType.DMA,
        pltpu.SemaphoreType.DMA,
    ],
    compiler_params=pltpu.CompilerParams(use_tc_tiling_on_sc=False),
)
def _embed_gather(table_hbm, idx_hbm, out_hbm, idx_v, rows0, rows1,
                  g0, g1, s0, s1):
    wid = lax.axis_index("s") * NUM_CORES + lax.axis_index("c")
    base = wid * BPW
    pltpu.sync_copy(idx_hbm.at[pl.ds(base, BPW)], idx_v)

    rows = (rows0, rows1)
    gsem = (g0, g1)
    ssem = (s0, s1)

    def gather(n, b):
        off = pl.multiple_of(n * CHUNK, CHUNK)
        pltpu.async_copy(table_hbm.at[idx_v.at[pl.ds(off, CHUNK)]],
                         rows[b], gsem[b])

    def store(n, b):
        off = pl.multiple_of(n * CHUNK, CHUNK)
        pltpu.async_copy(
            rows[b],
            out_hbm.at[pl.ds(base + off, CHUNK), pl.ds(0, EMBED_DIM)],
            ssem[b])

    gather(0, 0)
    gather(1, 1)

    def body(i, carry):
        for b in range(2):
            n = 2 * i + b
            pltpu.make_async_copy(table_hbm.at[idx_v.at[pl.ds(0, CHUNK)]],
                                  rows[b], gsem[b]).wait()
            store(n, b)
            pltpu.make_async_copy(
                rows[b],
                out_hbm.at[pl.ds(0, CHUNK), pl.ds(0, EMBED_DIM)],
                ssem[b]).wait()

            @pl.when(n + 2 < NCHUNK)
            def _():
                gather(n + 2, b)

        return carry

    lax.fori_loop(0, NCHUNK // 2, body, 0)


def kernel(input_tensor, table):
    tpad = jnp.pad(table, ((0, 0), (0, PADDED - EMBED_DIM)))
    t2 = tpad.reshape(2 * VOCAB, EMBED_DIM)
    idxt = input_tensor.T.reshape(-1) * 2
    out = _embed_gather(t2, idxt)
    out = out.reshape(HIST, BATCH, PADDED)[:, :, :EMBED_DIM]
    return out.transpose(1, 0, 2)


# speedup vs baseline: 7.7638x; 1.0043x over previous
import functools

import jax
import jax.numpy as jnp
from jax import lax
from jax.experimental import pallas as pl
from jax.experimental.pallas import tpu as pltpu
from jax.experimental.pallas import tpu_sc as plsc

VOCAB = 100000
EMBED_DIM = 64
PADDED = 128
BATCH = 4096
HIST = 50

TOTAL = BATCH * HIST
NUM_CORES = 2
NUM_SUBCORES = 16
NW = NUM_CORES * NUM_SUBCORES
BPW = TOTAL // NW
CHUNK = 400
NCHUNK = BPW // CHUNK

_mesh = plsc.VectorSubcoreMesh(core_axis_name="c", subcore_axis_name="s")


@functools.partial(
    pl.kernel,
    out_type=jax.ShapeDtypeStruct((TOTAL, PADDED), jnp.float32),
    mesh=_mesh,
    scratch_types=[
        pltpu.VMEM((BPW,), jnp.int32),
        pltpu.VMEM((CHUNK, EMBED_DIM), jnp.float32),
        pltpu.VMEM((CHUNK, EMBED_DIM), jnp.float32),
        pltpu.VMEM((CHUNK, EMBED_DIM), jnp.float32),
        pltpu.VMEM((CHUNK, EMBED_DIM), jnp.float32),
        pltpu.SemaphoreType.DMA,
        pltpu.SemaphoreType.DMA,
        pltpu.SemaphoreType.DMA,
        pltpu.SemaphoreType.DMA,
        pltpu.SemaphoreType.DMA,
        pltpu.SemaphoreType.DMA,
        pltpu.SemaphoreType.DMA,
        pltpu.SemaphoreType.DMA,
    ],
    compiler_params=pltpu.CompilerParams(use_tc_tiling_on_sc=False),
)
def _embed_gather(table_hbm, idx_hbm, out_hbm, idx_v, rows0, rows1, rows2,
                  rows3, g0, g1, g2, g3, s0, s1, s2, s3):
    wid = lax.axis_index("s") * NUM_CORES + lax.axis_index("c")
    base = wid * BPW
    pltpu.sync_copy(idx_hbm.at[pl.ds(base, BPW)], idx_v)

    rows = (rows0, rows1, rows2, rows3)
    gsem = (g0, g1, g2, g3)
    ssem = (s0, s1, s2, s3)

    def gather(n, b):
        off = pl.multiple_of(n * CHUNK, CHUNK)
        pltpu.async_copy(table_hbm.at[idx_v.at[pl.ds(off, CHUNK)]],
                         rows[b], gsem[b])

    def store(n, b):
        off = pl.multiple_of(n * CHUNK, CHUNK)
        pltpu.async_copy(
            rows[b],
            out_hbm.at[pl.ds(base + off, CHUNK), pl.ds(0, EMBED_DIM)],
            ssem[b])

    for b in range(4):
        gather(b, b)

    def wait_gather(b):
        pltpu.make_async_copy(table_hbm.at[idx_v.at[pl.ds(0, CHUNK)]],
                              rows[b], gsem[b]).wait()

    def wait_store(b):
        pltpu.make_async_copy(
            rows[b],
            out_hbm.at[pl.ds(0, CHUNK), pl.ds(0, EMBED_DIM)],
            ssem[b]).wait()

    def body(i, carry):
        for b in range(4):
            n = 4 * i + b
            wait_gather(b)
            store(n, b)

            @pl.when(n + 4 < NCHUNK)
            def _():
                wait_store(b)
                gather(n + 4, b)

        return carry

    lax.fori_loop(0, NCHUNK // 4, body, 0)


def kernel(input_tensor, table):
    tpad = jnp.pad(table, ((0, 0), (0, PADDED - EMBED_DIM)))
    t2 = tpad.reshape(2 * VOCAB, EMBED_DIM)
    idxt = input_tensor.T.reshape(-1) * 2
    out = _embed_gather(t2, idxt)
    out = out.reshape(HIST, BATCH, PADDED)[:, :, :EMBED_DIM]
    return out.transpose(1, 0, 2)
